# baseline (speedup 1.0000x reference)
import functools

import jax
import jax.numpy as jnp
from jax import lax
from jax.experimental import pallas as pl
from jax.experimental.pallas import tpu as pltpu
from jax.experimental.pallas import tpu_sc as plsc

N = 10000
E = 320000
NFEAT = 128
NHID = 128
NCLASS = 16
STEP = 0.1

NC = 2
NS = 16
NW = NC * NS

EPAD = 327680
NPAD = 10240
RPT = NPAD // NS

DEC = 128
DG = 8
DNGRP = EPAD // (DG * DEC)
DNG = DNGRP // NW

EC = 16
G = 8
NGRP = EPAD // (G * EC)
NG_SP = NGRP // NS

TROWS = 10112
TRPT = TROWS // NS
HALF = NPAD // 2
GARB = HALF
AROWS = HALF + NS

_mesh = plsc.VectorSubcoreMesh(core_axis_name="c", subcore_axis_name="s")


@functools.partial(
    pl.kernel,
    out_type=jax.ShapeDtypeStruct((NC, NPAD), jnp.float32),
    mesh=_mesh,
    scratch_types=[
        pltpu.VMEM((DG, DEC), jnp.int32),
        pltpu.VMEM((RPT,), jnp.float32),
        pltpu.VMEM((DEC,), jnp.float32),
        pltpu.VMEM_SHARED((NPAD,), jnp.float32),
    ],
)
def _deg_kernel(dst_hbm, out_hbm, dst_v, zero_v, ones_v, deg_sh):
    c = lax.axis_index("c")
    s = lax.axis_index("s")
    gid = c * NS + s

    def fill(i, _):
        zero_v[pl.ds(i * 16, 16)] = jnp.zeros((16,), jnp.float32)
        return _

    lax.fori_loop(0, RPT // 16, fill, 0)
    for j in range(DEC // 16):
        ones_v[pl.ds(j * 16, 16)] = jnp.full((16,), 1.0, jnp.float32)

    pltpu.sync_copy(zero_v, deg_sh.at[pl.ds(s * RPT, RPT)])
    plsc.subcore_barrier()

    def gstep(g, _):
        pltpu.sync_copy(dst_hbm.at[gid * DNG + g], dst_v)
        for k in range(DG):
            pltpu.sync_copy(ones_v, deg_sh.at[dst_v.at[k]], add=True)
        return _

    lax.fori_loop(0, DNG, gstep, 0)
    plsc.subcore_barrier()
    pltpu.sync_copy(deg_sh.at[pl.ds(s * RPT, RPT)],
                    out_hbm.at[c, pl.ds(s * RPT, RPT)])


@functools.partial(
    pl.kernel,
    out_type=jax.ShapeDtypeStruct((NC, HALF, NHID), jnp.float32),
    mesh=_mesh,
    scratch_types=[
        pltpu.VMEM((G, EC), jnp.int32),
        pltpu.VMEM((G, EC), jnp.int32),
        pltpu.VMEM((EC, NHID), jnp.float32),
        pltpu.VMEM_SHARED((TROWS, NHID), jnp.float32),
        pltpu.VMEM_SHARED((AROWS, NHID), jnp.float32),
        pltpu.SemaphoreType.DMA,
    ],
)
def _spmm_kernel(mp_hbm, src_hbm, dst0_hbm, dst1_hbm, out_hbm,
                 src_v, dst_v, rows_v, tab_sh, acc_sh, sem):
    c = lax.axis_index("c")
    s = lax.axis_index("s")

    def fill(i, _):
        r = i // (NHID // 16)
        j = i - r * (NHID // 16)
        rows_v[r, pl.ds(j * 16, 16)] = jnp.zeros((16,), jnp.float32)
        return _

    lax.fori_loop(0, EC * (NHID // 16), fill, 0)
    for b in range(HALF // NS // EC):
        r = s * (HALF // NS) + b * EC
        pltpu.sync_copy(rows_v, acc_sh.at[pl.ds(r, EC)])

    @pl.when(s == 0)
    def _zero_tail():
        pltpu.sync_copy(rows_v, acc_sh.at[pl.ds(HALF, AROWS - HALF)])
    pltpu.sync_copy(mp_hbm.at[pl.ds(s * TRPT, TRPT)],
                    tab_sh.at[pl.ds(s * TRPT, TRPT)])
    plsc.subcore_barrier()

    def body(dst_grp_hbm):
        def gstep(g, _):
            pltpu.sync_copy(src_hbm.at[s * NG_SP + g], src_v)
            pltpu.sync_copy(dst_grp_hbm.at[s * NG_SP + g], dst_v)
            for k in range(G):
                pltpu.async_copy(tab_sh.at[src_v.at[k]], rows_v, sem).wait()
                pltpu.sync_copy(rows_v, acc_sh.at[dst_v.at[k]], add=True)
            return _

        lax.fori_loop(0, NG_SP, gstep, 0)

    @pl.when(c == 0)
    def _loop0():
        body(dst0_hbm)

    @pl.when(c != 0)
    def _loop1():
        body(dst1_hbm)

    plsc.subcore_barrier()
    for b in range(HALF // NS // EC):
        r = s * (HALF // NS) + b * EC
        pltpu.sync_copy(acc_sh.at[pl.ds(r, EC)], out_hbm.at[c, pl.ds(r, EC)])


BLK = 2048


def _idx_body(dst_ref, dst0_ref, dst1_ref):
    d = dst_ref[...]
    dst0_ref[...] = jnp.where(d < HALF, d, GARB)
    dst1_ref[...] = jnp.where(d >= HALF, d - HALF, GARB)


def _enc_body(x_ref, we_ref, om_ref, deg_ref, h_ref, mp_ref, dinv_ref):
    h = jnp.dot(x_ref[...], we_ref[...], preferred_element_type=jnp.float32)
    d = deg_ref[...]
    dinv = lax.rsqrt(d[:, 0:1] + d[:, 1:2] + 1.0)
    om = om_ref[...]
    wsym = 0.5 * (om + om.T)
    m = jnp.dot(h, wsym, preferred_element_type=jnp.float32)
    h_ref[...] = h
    mp_ref[...] = dinv * m
    dinv_ref[...] = dinv


def _layer_body(h_ref, h0_ref, p_ref, mp_ref, dinv_ref, wext_ref, beta_ref,
                om_ref, hn_ref, mpn_ref):
    h = h_ref[...]
    mp = mp_ref[...]
    dinv = dinv_ref[...]
    agg = dinv * (p_ref[...] + mp)
    dh = agg - h * wext_ref[...] - beta_ref[0, 0] * h0_ref[...]
    hn = h + STEP * dh
    om = om_ref[...]
    wsym = 0.5 * (om + om.T)
    hn_ref[...] = hn
    mpn_ref[...] = dinv * jnp.dot(hn, wsym, preferred_element_type=jnp.float32)


def _final_body(h_ref, h0_ref, p_ref, mp_ref, dinv_ref, wext_ref, beta_ref,
                wd_ref, out_ref):
    h = h_ref[...]
    agg = dinv_ref[...] * (p_ref[...] + mp_ref[...])
    dh = agg - h * wext_ref[...] - beta_ref[0, 0] * h0_ref[...]
    hn = h + STEP * dh
    o = jnp.dot(hn, wd_ref[...], preferred_element_type=jnp.float32)
    m = jnp.max(o, axis=1, keepdims=True)
    e = o - m
    lse = jnp.log(jnp.sum(jnp.exp(e), axis=1, keepdims=True))
    out_ref[...] = e - lse


def _rows(i):
    return (i, 0)


def _fixed(i):
    return (0, 0)


def _rows_spec(w):
    return pl.BlockSpec((BLK, w), _rows)


def _full_spec(a, b):
    return pl.BlockSpec((a, b), _fixed)


_GRID = NPAD // BLK


def _idx_call(dst_flat):
    rows = EPAD // 512
    spec = pl.BlockSpec((rows // 4, 512), lambda i: (i, 0))
    d0, d1 = pl.pallas_call(
        _idx_body,
        grid=(4,),
        in_specs=[spec],
        out_specs=[spec, spec],
        out_shape=[jax.ShapeDtypeStruct((rows, 512), jnp.int32),
                   jax.ShapeDtypeStruct((rows, 512), jnp.int32)],
    )(dst_flat.reshape(rows, 512))
    return d0.reshape(EPAD), d1.reshape(EPAD)


def _enc_call(xp, w_enc, omega, degp_t):
    return pl.pallas_call(
        _enc_body,
        grid=(_GRID,),
        in_specs=[_rows_spec(NFEAT), _full_spec(NFEAT, NHID),
                  _full_spec(NHID, NHID), _rows_spec(2)],
        out_specs=[_rows_spec(NHID), _rows_spec(NHID), _rows_spec(1)],
        out_shape=[jax.ShapeDtypeStruct((NPAD, NHID), jnp.float32),
                   jax.ShapeDtypeStruct((NPAD, NHID), jnp.float32),
                   jax.ShapeDtypeStruct((NPAD, 1), jnp.float32)],
    )(xp, w_enc, omega, degp_t)


def _layer_call(h, h0, s_agg, mp, dinv, wext2, beta2, omega):
    return pl.pallas_call(
        _layer_body,
        grid=(_GRID,),
        in_specs=[_rows_spec(NHID), _rows_spec(NHID), _rows_spec(NHID),
                  _rows_spec(NHID), _rows_spec(1), _full_spec(1, NHID),
                  _full_spec(1, 1), _full_spec(NHID, NHID)],
        out_specs=[_rows_spec(NHID), _rows_spec(NHID)],
        out_shape=[jax.ShapeDtypeStruct((NPAD, NHID), jnp.float32),
                   jax.ShapeDtypeStruct((NPAD, NHID), jnp.float32)],
    )(h, h0, s_agg, mp, dinv, wext2, beta2, omega)


def _final_call(h, h0, s_agg, mp, dinv, wext2, beta2, w_dec):
    return pl.pallas_call(
        _final_body,
        grid=(_GRID,),
        in_specs=[_rows_spec(NHID), _rows_spec(NHID), _rows_spec(NHID),
                  _rows_spec(NHID), _rows_spec(1), _full_spec(1, NHID),
                  _full_spec(1, 1), _full_spec(NHID, NCLASS)],
        out_specs=_rows_spec(NCLASS),
        out_shape=jax.ShapeDtypeStruct((NPAD, NCLASS), jnp.float32),
    )(h, h0, s_agg, mp, dinv, wext2, beta2, w_dec)


@jax.jit
def kernel(x, adj, w_enc, w_ext, omega, beta, w_dec):
    src = adj[0]
    dst = adj[1]
    pad = jnp.full((EPAD - E,), N, jnp.int32)
    src_flat = jnp.concatenate([src, pad])
    dst_flat = jnp.concatenate([dst, pad])
    dst0_flat, dst1_flat = _idx_call(dst_flat)
    src_p = src_flat.reshape(NGRP, G, EC)
    dst0_p = dst0_flat.reshape(NGRP, G, EC)
    dst1_p = dst1_flat.reshape(NGRP, G, EC)
    dstd_p = dst_flat.reshape(DNGRP, DG, DEC)
    xp = jnp.zeros((NPAD, NFEAT), jnp.float32).at[:N].set(x)
    wext2 = w_ext.reshape(1, NHID)
    beta2 = beta.reshape(1, 1)

    degp = _deg_kernel(dstd_p)
    h, mp1, dinv = _enc_call(xp, w_enc, omega, degp.T)
    parts1 = _spmm_kernel(mp1, src_p, dst0_p, dst1_p)
    s1 = parts1.reshape(NPAD, NHID)
    h1, mp2 = _layer_call(h, h, s1, mp1, dinv, wext2, beta2, omega)
    parts2 = _spmm_kernel(mp2, src_p, dst0_p, dst1_p)
    s2 = parts2.reshape(NPAD, NHID)
    out = _final_call(h1, h, s2, mp2, dinv, wext2, beta2, w_dec)
    return out[:N]


# speedup vs baseline: 1.2941x; 1.2941x over previous
import functools

import jax
import jax.numpy as jnp
from jax import lax
from jax.experimental import pallas as pl
from jax.experimental.pallas import tpu as pltpu
from jax.experimental.pallas import tpu_sc as plsc

N = 10000
E = 320000
NFEAT = 128
NHID = 128
NCLASS = 16
STEP = 0.1

NC = 2
NS = 16
NW = NC * NS

EPAD = 327680
NPAD = 10240
RPT = NPAD // NS

DEC = 128
DG = 8
DNGRP = EPAD // (DG * DEC)
DNG = DNGRP // NW

EC = 16
G = 8
NGRP = EPAD // (G * EC)
NG_SP = NGRP // NS

TROWS = 10112
TRPT = TROWS // NS
HALF = NPAD // 2
GARB = HALF
AROWS = HALF + NS

_mesh = plsc.VectorSubcoreMesh(core_axis_name="c", subcore_axis_name="s")


@functools.partial(
    pl.kernel,
    out_type=jax.ShapeDtypeStruct((NC, NPAD), jnp.float32),
    mesh=_mesh,
    scratch_types=[
        pltpu.VMEM((DG, DEC), jnp.int32),
        pltpu.VMEM((RPT,), jnp.float32),
        pltpu.VMEM((DEC,), jnp.float32),
        pltpu.VMEM_SHARED((NPAD,), jnp.float32),
    ],
)
def _deg_kernel(dst_hbm, out_hbm, dst_v, zero_v, ones_v, deg_sh):
    c = lax.axis_index("c")
    s = lax.axis_index("s")
    gid = c * NS + s

    def fill(i, _):
        zero_v[pl.ds(i * 16, 16)] = jnp.zeros((16,), jnp.float32)
        return _

    lax.fori_loop(0, RPT // 16, fill, 0)
    for j in range(DEC // 16):
        ones_v[pl.ds(j * 16, 16)] = jnp.full((16,), 1.0, jnp.float32)

    pltpu.sync_copy(zero_v, deg_sh.at[pl.ds(s * RPT, RPT)])
    plsc.subcore_barrier()

    def gstep(g, _):
        pltpu.sync_copy(dst_hbm.at[gid * DNG + g], dst_v)
        for k in range(DG):
            pltpu.sync_copy(ones_v, deg_sh.at[dst_v.at[k]], add=True)
        return _

    lax.fori_loop(0, DNG, gstep, 0)
    plsc.subcore_barrier()
    pltpu.sync_copy(deg_sh.at[pl.ds(s * RPT, RPT)],
                    out_hbm.at[c, pl.ds(s * RPT, RPT)])


@functools.partial(
    pl.kernel,
    out_type=jax.ShapeDtypeStruct((NC, HALF, NHID), jnp.float32),
    mesh=_mesh,
    scratch_types=[
        pltpu.VMEM((G, EC), jnp.int32),
        pltpu.VMEM((G, EC), jnp.int32),
        pltpu.VMEM((EC, NHID), jnp.float32),
        pltpu.VMEM((EC, NHID), jnp.float32),
        pltpu.VMEM_SHARED((TROWS, NHID), jnp.float32),
        pltpu.VMEM_SHARED((AROWS, NHID), jnp.float32),
        pltpu.SemaphoreType.DMA,
        pltpu.SemaphoreType.DMA,
    ],
)
def _spmm_kernel(mp_hbm, src_hbm, dst0_hbm, dst1_hbm, out_hbm,
                 src_v, dst_v, rows_v, rows_b, tab_sh, acc_sh, sem, sem_b):
    c = lax.axis_index("c")
    s = lax.axis_index("s")

    def fill(i, _):
        r = i // (NHID // 16)
        j = i - r * (NHID // 16)
        rows_v[r, pl.ds(j * 16, 16)] = jnp.zeros((16,), jnp.float32)
        return _

    lax.fori_loop(0, EC * (NHID // 16), fill, 0)
    for b in range(HALF // NS // EC):
        r = s * (HALF // NS) + b * EC
        pltpu.sync_copy(rows_v, acc_sh.at[pl.ds(r, EC)])

    @pl.when(s == 0)
    def _zero_tail():
        pltpu.sync_copy(rows_v, acc_sh.at[pl.ds(HALF, AROWS - HALF)])
    pltpu.sync_copy(mp_hbm.at[pl.ds(s * TRPT, TRPT)],
                    tab_sh.at[pl.ds(s * TRPT, TRPT)])
    plsc.subcore_barrier()

    def body(dst_grp_hbm):
        def gstep(g, _):
            pltpu.sync_copy(src_hbm.at[s * NG_SP + g], src_v)
            pltpu.sync_copy(dst_grp_hbm.at[s * NG_SP + g], dst_v)
            cps = {0: pltpu.async_copy(tab_sh.at[src_v.at[0]], rows_v, sem)}
            for k in range(G):
                buf = rows_v if k % 2 == 0 else rows_b
                if k + 1 < G:
                    nbuf = rows_b if k % 2 == 0 else rows_v
                    nsem = sem_b if k % 2 == 0 else sem
                    cps[k + 1] = pltpu.async_copy(
                        tab_sh.at[src_v.at[k + 1]], nbuf, nsem)
                cps[k].wait()
                pltpu.sync_copy(buf, acc_sh.at[dst_v.at[k]], add=True)
            return _

        lax.fori_loop(0, NG_SP, gstep, 0)

    @pl.when(c == 0)
    def _loop0():
        body(dst0_hbm)

    @pl.when(c != 0)
    def _loop1():
        body(dst1_hbm)

    plsc.subcore_barrier()
    for b in range(HALF // NS // EC):
        r = s * (HALF // NS) + b * EC
        pltpu.sync_copy(acc_sh.at[pl.ds(r, EC)], out_hbm.at[c, pl.ds(r, EC)])


BLK = 2048


def _idx_body(dst_ref, dst0_ref, dst1_ref):
    d = dst_ref[...]
    dst0_ref[...] = jnp.where(d < HALF, d, GARB)
    dst1_ref[...] = jnp.where(d >= HALF, d - HALF, GARB)


def _enc_body(x_ref, we_ref, om_ref, deg_ref, h_ref, mp_ref, dinv_ref):
    h = jnp.dot(x_ref[...], we_ref[...], preferred_element_type=jnp.float32)
    d = deg_ref[...]
    dinv = lax.rsqrt(d[:, 0:1] + d[:, 1:2] + 1.0)
    om = om_ref[...]
    wsym = 0.5 * (om + om.T)
    m = jnp.dot(h, wsym, preferred_element_type=jnp.float32)
    h_ref[...] = h
    mp_ref[...] = dinv * m
    dinv_ref[...] = dinv


def _layer_body(h_ref, h0_ref, p_ref, mp_ref, dinv_ref, wext_ref, beta_ref,
                om_ref, hn_ref, mpn_ref):
    h = h_ref[...]
    mp = mp_ref[...]
    dinv = dinv_ref[...]
    agg = dinv * (p_ref[...] + mp)
    dh = agg - h * wext_ref[...] - beta_ref[0, 0] * h0_ref[...]
    hn = h + STEP * dh
    om = om_ref[...]
    wsym = 0.5 * (om + om.T)
    hn_ref[...] = hn
    mpn_ref[...] = dinv * jnp.dot(hn, wsym, preferred_element_type=jnp.float32)


def _final_body(h_ref, h0_ref, p_ref, mp_ref, dinv_ref, wext_ref, beta_ref,
                wd_ref, out_ref):
    h = h_ref[...]
    agg = dinv_ref[...] * (p_ref[...] + mp_ref[...])
    dh = agg - h * wext_ref[...] - beta_ref[0, 0] * h0_ref[...]
    hn = h + STEP * dh
    o = jnp.dot(hn, wd_ref[...], preferred_element_type=jnp.float32)
    m = jnp.max(o, axis=1, keepdims=True)
    e = o - m
    lse = jnp.log(jnp.sum(jnp.exp(e), axis=1, keepdims=True))
    out_ref[...] = e - lse


def _rows(i):
    return (i, 0)


def _fixed(i):
    return (0, 0)


def _rows_spec(w):
    return pl.BlockSpec((BLK, w), _rows)


def _full_spec(a, b):
    return pl.BlockSpec((a, b), _fixed)


_GRID = NPAD // BLK


def _idx_call(dst_flat):
    rows = EPAD // 512
    spec = pl.BlockSpec((rows // 4, 512), lambda i: (i, 0))
    d0, d1 = pl.pallas_call(
        _idx_body,
        grid=(4,),
        in_specs=[spec],
        out_specs=[spec, spec],
        out_shape=[jax.ShapeDtypeStruct((rows, 512), jnp.int32),
                   jax.ShapeDtypeStruct((rows, 512), jnp.int32)],
    )(dst_flat.reshape(rows, 512))
    return d0.reshape(EPAD), d1.reshape(EPAD)


def _enc_call(xp, w_enc, omega, degp_t):
    return pl.pallas_call(
        _enc_body,
        grid=(_GRID,),
        in_specs=[_rows_spec(NFEAT), _full_spec(NFEAT, NHID),
                  _full_spec(NHID, NHID), _rows_spec(2)],
        out_specs=[_rows_spec(NHID), _rows_spec(NHID), _rows_spec(1)],
        out_shape=[jax.ShapeDtypeStruct((NPAD, NHID), jnp.float32),
                   jax.ShapeDtypeStruct((NPAD, NHID), jnp.float32),
                   jax.ShapeDtypeStruct((NPAD, 1), jnp.float32)],
    )(xp, w_enc, omega, degp_t)


def _layer_call(h, h0, s_agg, mp, dinv, wext2, beta2, omega):
    return pl.pallas_call(
        _layer_body,
        grid=(_GRID,),
        in_specs=[_rows_spec(NHID), _rows_spec(NHID), _rows_spec(NHID),
                  _rows_spec(NHID), _rows_spec(1), _full_spec(1, NHID),
                  _full_spec(1, 1), _full_spec(NHID, NHID)],
        out_specs=[_rows_spec(NHID), _rows_spec(NHID)],
        out_shape=[jax.ShapeDtypeStruct((NPAD, NHID), jnp.float32),
                   jax.ShapeDtypeStruct((NPAD, NHID), jnp.float32)],
    )(h, h0, s_agg, mp, dinv, wext2, beta2, omega)


def _final_call(h, h0, s_agg, mp, dinv, wext2, beta2, w_dec):
    return pl.pallas_call(
        _final_body,
        grid=(_GRID,),
        in_specs=[_rows_spec(NHID), _rows_spec(NHID), _rows_spec(NHID),
                  _rows_spec(NHID), _rows_spec(1), _full_spec(1, NHID),
                  _full_spec(1, 1), _full_spec(NHID, NCLASS)],
        out_specs=_rows_spec(NCLASS),
        out_shape=jax.ShapeDtypeStruct((NPAD, NCLASS), jnp.float32),
    )(h, h0, s_agg, mp, dinv, wext2, beta2, w_dec)


@jax.jit
def kernel(x, adj, w_enc, w_ext, omega, beta, w_dec):
    src = adj[0]
    dst = adj[1]
    pad = jnp.full((EPAD - E,), N, jnp.int32)
    src_flat = jnp.concatenate([src, pad])
    dst_flat = jnp.concatenate([dst, pad])
    dst0_flat, dst1_flat = _idx_call(dst_flat)
    src_p = src_flat.reshape(NGRP, G, EC)
    dst0_p = dst0_flat.reshape(NGRP, G, EC)
    dst1_p = dst1_flat.reshape(NGRP, G, EC)
    dstd_p = dst_flat.reshape(DNGRP, DG, DEC)
    xp = jnp.zeros((NPAD, NFEAT), jnp.float32).at[:N].set(x)
    wext2 = w_ext.reshape(1, NHID)
    beta2 = beta.reshape(1, 1)

    degp = _deg_kernel(dstd_p)
    h, mp1, dinv = _enc_call(xp, w_enc, omega, degp.T)
    parts1 = _spmm_kernel(mp1, src_p, dst0_p, dst1_p)
    s1 = parts1.reshape(NPAD, NHID)
    h1, mp2 = _layer_call(h, h, s1, mp1, dinv, wext2, beta2, omega)
    parts2 = _spmm_kernel(mp2, src_p, dst0_p, dst1_p)
    s2 = parts2.reshape(NPAD, NHID)
    out = _final_call(h1, h, s2, mp2, dinv, wext2, beta2, w_dec)
    return out[:N]


# speedup vs baseline: 1.7053x; 1.3178x over previous
import functools

import jax
import jax.numpy as jnp
from jax import lax
from jax.experimental import pallas as pl
from jax.experimental.pallas import tpu as pltpu
from jax.experimental.pallas import tpu_sc as plsc

N = 10000
E = 320000
NFEAT = 128
NHID = 128
NCLASS = 16
STEP = 0.1

NC = 2
NS = 16
NW = NC * NS

LANE = 128
CH = 79
EPT = CH * LANE
EPAD = NW * EPT
NPAD = 10240
RPT = NPAD // NS

_mesh = plsc.VectorSubcoreMesh(core_axis_name="c", subcore_axis_name="s")


@functools.partial(
    pl.kernel,
    out_type=jax.ShapeDtypeStruct((NC, NPAD), jnp.float32),
    mesh=_mesh,
    scratch_types=[
        pltpu.VMEM((CH, LANE), jnp.int32),
        pltpu.VMEM((RPT,), jnp.float32),
        pltpu.VMEM((LANE,), jnp.float32),
        pltpu.VMEM_SHARED((NPAD,), jnp.float32),
    ],
)
def _deg_kernel(dst_hbm, out_hbm, dst_v, zero_v, ones_v, deg_sh):
    c = lax.axis_index("c")
    s = lax.axis_index("s")
    gid = c * NS + s

    def fill(i, _):
        zero_v[pl.ds(i * 16, 16)] = jnp.zeros((16,), jnp.float32)
        return _

    lax.fori_loop(0, RPT // 16, fill, 0)
    for j in range(LANE // 16):
        ones_v[pl.ds(j * 16, 16)] = jnp.full((16,), 1.0, jnp.float32)

    pltpu.sync_copy(zero_v, deg_sh.at[pl.ds(s * RPT, RPT)])
    plsc.subcore_barrier()

    pltpu.sync_copy(dst_hbm.at[gid], dst_v)

    def step(ch, _):
        pltpu.sync_copy(ones_v, deg_sh.at[dst_v.at[ch]], add=True)
        return _

    lax.fori_loop(0, CH, step, 0)
    plsc.subcore_barrier()
    pltpu.sync_copy(deg_sh.at[pl.ds(s * RPT, RPT)],
                    out_hbm.at[c, pl.ds(s * RPT, RPT)])


@functools.partial(
    pl.kernel,
    out_type=jax.ShapeDtypeStruct((NC, NPAD, NHID), jnp.float32),
    mesh=_mesh,
    scratch_types=[
        pltpu.VMEM((CH, LANE), jnp.int32),
        pltpu.VMEM((CH, LANE), jnp.int32),
        pltpu.VMEM((LANE, NHID), jnp.float32),
        pltpu.VMEM_SHARED((NPAD, NHID), jnp.float32),
        pltpu.SemaphoreType.DMA,
    ],
)
def _spmm_kernel(table_hbm, src_hbm, dst_hbm, out_hbm,
                 src_v, dst_v, rows_v, acc_sh, gsem):
    c = lax.axis_index("c")
    s = lax.axis_index("s")
    gid = c * NS + s

    def fill(i, _):
        r = i // (NHID // 16)
        j = i - r * (NHID // 16)
        rows_v[r, pl.ds(j * 16, 16)] = jnp.zeros((16,), jnp.float32)
        return _

    lax.fori_loop(0, LANE * (NHID // 16), fill, 0)
    for b in range(RPT // LANE):
        pltpu.sync_copy(rows_v, acc_sh.at[pl.ds(s * RPT + b * LANE, LANE)])
    plsc.subcore_barrier()

    pltpu.sync_copy(src_hbm.at[gid], src_v)
    pltpu.sync_copy(dst_hbm.at[gid], dst_v)

    def step(ch, _):
        pltpu.async_copy(table_hbm.at[src_v.at[ch]], rows_v, gsem).wait()
        pltpu.sync_copy(rows_v, acc_sh.at[dst_v.at[ch]], add=True)
        return _

    lax.fori_loop(0, CH, step, 0)
    plsc.subcore_barrier()
    for b in range(RPT // LANE):
        r = s * RPT + b * LANE
        pltpu.sync_copy(acc_sh.at[pl.ds(r, LANE)], out_hbm.at[c, pl.ds(r, LANE)])


BLK = 2048


def _enc_body(x_ref, we_ref, om_ref, deg_ref, h_ref, mp_ref, dinv_ref):
    h = jnp.dot(x_ref[...], we_ref[...], preferred_element_type=jnp.float32)
    d = deg_ref[...]
    dinv = lax.rsqrt(d[:, 0:1] + d[:, 1:2] + 1.0)
    om = om_ref[...]
    wsym = 0.5 * (om + om.T)
    m = jnp.dot(h, wsym, preferred_element_type=jnp.float32)
    h_ref[...] = h
    mp_ref[...] = dinv * m
    dinv_ref[...] = dinv


def _layer_body(h_ref, h0_ref, p_ref, mp_ref, dinv_ref, wext_ref, beta_ref,
                om_ref, hn_ref, mpn_ref):
    h = h_ref[...]
    mp = mp_ref[...]
    dinv = dinv_ref[...]
    agg = dinv * (p_ref[0] + p_ref[1] + mp)
    dh = agg - h * wext_ref[...] - beta_ref[0, 0] * h0_ref[...]
    hn = h + STEP * dh
    om = om_ref[...]
    wsym = 0.5 * (om + om.T)
    hn_ref[...] = hn
    mpn_ref[...] = dinv * jnp.dot(hn, wsym, preferred_element_type=jnp.float32)


def _final_body(h_ref, h0_ref, p_ref, mp_ref, dinv_ref, wext_ref, beta_ref,
                wd_ref, out_ref):
    h = h_ref[...]
    agg = dinv_ref[...] * (p_ref[0] + p_ref[1] + mp_ref[...])
    dh = agg - h * wext_ref[...] - beta_ref[0, 0] * h0_ref[...]
    hn = h + STEP * dh
    o = jnp.dot(hn, wd_ref[...], preferred_element_type=jnp.float32)
    m = jnp.max(o, axis=1, keepdims=True)
    e = o - m
    lse = jnp.log(jnp.sum(jnp.exp(e), axis=1, keepdims=True))
    out_ref[...] = e - lse


def _rows(i):
    return (i, 0)


def _fixed(i):
    return (0, 0)


def _rows_spec(w):
    return pl.BlockSpec((BLK, w), _rows)


def _full_spec(a, b):
    return pl.BlockSpec((a, b), _fixed)


_GRID = NPAD // BLK


def _enc_call(xp, w_enc, omega, degp_t):
    return pl.pallas_call(
        _enc_body,
        grid=(_GRID,),
        in_specs=[_rows_spec(NFEAT), _full_spec(NFEAT, NHID),
                  _full_spec(NHID, NHID), _rows_spec(2)],
        out_specs=[_rows_spec(NHID), _rows_spec(NHID), _rows_spec(1)],
        out_shape=[jax.ShapeDtypeStruct((NPAD, NHID), jnp.float32),
                   jax.ShapeDtypeStruct((NPAD, NHID), jnp.float32),
                   jax.ShapeDtypeStruct((NPAD, 1), jnp.float32)],
    )(xp, w_enc, omega, degp_t)


def _layer_call(h, h0, parts, mp, dinv, wext2, beta2, omega):
    pspec = pl.BlockSpec((NC, BLK, NHID), lambda i: (0, i, 0))
    return pl.pallas_call(
        _layer_body,
        grid=(_GRID,),
        in_specs=[_rows_spec(NHID), _rows_spec(NHID), pspec, _rows_spec(NHID),
                  _rows_spec(1), _full_spec(1, NHID), _full_spec(1, 1),
                  _full_spec(NHID, NHID)],
        out_specs=[_rows_spec(NHID), _rows_spec(NHID)],
        out_shape=[jax.ShapeDtypeStruct((NPAD, NHID), jnp.float32),
                   jax.ShapeDtypeStruct((NPAD, NHID), jnp.float32)],
    )(h, h0, parts, mp, dinv, wext2, beta2, omega)


def _final_call(h, h0, parts, mp, dinv, wext2, beta2, w_dec):
    pspec = pl.BlockSpec((NC, BLK, NHID), lambda i: (0, i, 0))
    return pl.pallas_call(
        _final_body,
        grid=(_GRID,),
        in_specs=[_rows_spec(NHID), _rows_spec(NHID), pspec, _rows_spec(NHID),
                  _rows_spec(1), _full_spec(1, NHID), _full_spec(1, 1),
                  _full_spec(NHID, NCLASS)],
        out_specs=_rows_spec(NCLASS),
        out_shape=jax.ShapeDtypeStruct((NPAD, NCLASS), jnp.float32),
    )(h, h0, parts, mp, dinv, wext2, beta2, w_dec)


@jax.jit
def kernel(x, adj, w_enc, w_ext, omega, beta, w_dec):
    src = adj[0]
    dst = adj[1]
    pad = jnp.full((EPAD - E,), N, jnp.int32)
    src_p = jnp.concatenate([src, pad]).reshape(NW, CH, LANE)
    dst_p = jnp.concatenate([dst, pad]).reshape(NW, CH, LANE)
    xp = jnp.zeros((NPAD, NFEAT), jnp.float32).at[:N].set(x)
    wext2 = w_ext.reshape(1, NHID)
    beta2 = beta.reshape(1, 1)

    degp = _deg_kernel(dst_p)
    h, mp1, dinv = _enc_call(xp, w_enc, omega, degp.T)
    parts1 = _spmm_kernel(mp1, src_p, dst_p)
    h1, mp2 = _layer_call(h, h, parts1, mp1, dinv, wext2, beta2, omega)
    parts2 = _spmm_kernel(mp2, src_p, dst_p)
    out = _final_call(h1, h, parts2, mp2, dinv, wext2, beta2, w_dec)
    return out[:N]
